# baseline (speedup 1.0000x reference)
import functools
import jax
import jax.numpy as jnp
from jax import lax
from jax.experimental import pallas as pl
from jax.experimental.pallas import tpu as pltpu
from jax.experimental.pallas import tpu_sc as plsc

_HD = 4096
_S = 2048
_NLAYERS = 3
_THRESH = 0.5
_KEEP = 1843
_LAM = 1.0
_INT_MIN = -2147483648

_NC = 2
_NS = 16
_LN = 16
_NW = _NC * _NS

_DCH = 64
_ECH = 4
_ACH = 8
_EROWS = _S // _NW
_AROWS = 304
_SC_AROWS = _NW * _AROWS


def _attn_body(a0_ref, a1_ref, a2_ref, cs_ref, *, n2):
    i = pl.program_id(0)

    @pl.when(i == 0)
    def _():
        cs_ref[...] = jnp.zeros_like(cs_ref)
    s = (jnp.sum(a0_ref[...], axis=0, keepdims=True)
         + jnp.sum(a1_ref[...], axis=0, keepdims=True))
    w = (i < n2).astype(jnp.float32)
    cs_ref[...] += s + w * jnp.sum(a2_ref[...], axis=0, keepdims=True)


def _tsum(xs):
    while len(xs) > 1:
        nx = [xs[p] + xs[p + 1] for p in range(0, len(xs) - 1, 2)]
        if len(xs) % 2:
            nx.append(xs[-1])
        xs = nx
    return xs[0]


def _sc_body(dec, emb, at2, hsum_o, cn_o, sq_o, ap_o,
             dbuf, ebuf, abuf, accv, hbuf, cnv, sqv, aacc, shsum,
             sem_a, sem_b):
    cid = lax.axis_index("c")
    sid = lax.axis_index("s")
    t = cid * _NS + sid
    sems = [sem_a, sem_b]

    ncols = _HD // _NS
    col0 = sid * ncols
    nch = _S // _DCH

    def _dstart(ci, b):
        pltpu.make_async_copy(
            dec.at[pl.ds(ci * _DCH, _DCH), pl.ds(col0, ncols)],
            dbuf.at[b], sems[b]).start()

    _dstart(0, 0)
    _dstart(1, 1)
    acc0 = tuple(jnp.zeros((_LN,), jnp.float32) for _ in range(ncols // _LN))

    def dchunk(c2, acc):
        for b in range(2):
            ci = 2 * c2 + b
            pltpu.make_async_copy(
                dec.at[pl.ds(ci * _DCH, _DCH), pl.ds(col0, ncols)],
                dbuf.at[b], sems[b]).wait()

            def rbody(r, a, b=b):
                return tuple(a[j] + dbuf[b, r, pl.ds(j * _LN, _LN)]
                             for j in range(ncols // _LN))
            acc = lax.fori_loop(0, _DCH, rbody, acc)

            @pl.when(ci + 2 < nch)
            def _(ci=ci, b=b):
                _dstart(ci + 2, b)
        return acc

    acc = lax.fori_loop(0, nch // 2, dchunk, acc0)

    for j in range(ncols // _LN):
        accv[pl.ds(j * _LN, _LN)] = acc[j]
    pltpu.sync_copy(accv, shsum.at[pl.ds(col0, ncols)])
    plsc.subcore_barrier()
    pltpu.sync_copy(shsum, hbuf)

    @pl.when(jnp.logical_and(cid == 0, sid == 0))
    def _():
        pltpu.sync_copy(hbuf, hsum_o.at[0])

    erow0 = t * _EROWS
    nech = _EROWS // _ECH
    lane_iota = lax.iota(jnp.int32, _LN)

    def _estart(ci, b):
        pltpu.make_async_copy(
            emb.at[pl.ds(erow0 + ci * _ECH, _ECH)],
            ebuf.at[b], sems[b]).start()

    _estart(0, 0)
    _estart(1, 1)

    def echunk(c2, _):
        for b in range(2):
            ci = 2 * c2 + b
            pltpu.make_async_copy(
                emb.at[pl.ds(erow0 + ci * _ECH, _ECH)],
                ebuf.at[b], sems[b]).wait()

            zc = jnp.zeros((_LN,), jnp.float32)

            @plsc.parallel_loop(0, _HD // 128, unroll=2,
                                carry=(zc,) * (2 * _ECH))
            def vs(bb, cs, b=b):
                out = list(cs)
                h8 = [hbuf[pl.ds(bb * 128 + u * _LN, _LN)] for u in range(8)]
                for r in range(_ECH):
                    ev = [ebuf[b, r, pl.ds(bb * 128 + u * _LN, _LN)]
                          for u in range(8)]
                    out[r] = out[r] + _tsum([ev[u] * h8[u] for u in range(8)])
                    out[_ECH + r] = out[_ECH + r] + _tsum(
                        [ev[u] * ev[u] for u in range(8)])
                return tuple(out)

            for r in range(_ECH):
                rl = jnp.full((_LN,), ci * _ECH + r, jnp.int32)
                plsc.store_scatter(cnv, [lane_iota, rl], vs[r])
                plsc.store_scatter(sqv, [lane_iota, rl], vs[_ECH + r])

            @pl.when(ci + 2 < nech)
            def _(ci=ci, b=b):
                _estart(ci + 2, b)
        return 0

    lax.fori_loop(0, nech // 2, echunk, 0)
    outcps = []
    for j in range(_LN):
        outcps.append(pltpu.make_async_copy(
            cnv.at[j], cn_o.at[pl.ds(j * _S + erow0, _EROWS)], sem_a))
        outcps.append(pltpu.make_async_copy(
            sqv.at[j], sq_o.at[pl.ds(j * _S + erow0, _EROWS)], sem_b))
    for cp in outcps:
        cp.start()
    for cp in outcps:
        cp.wait()

    arow0 = t * _AROWS
    zv = jnp.zeros((_LN,), jnp.float32)

    def zbody(j, _):
        aacc[pl.ds(j * _LN, _LN)] = zv
        return 0
    lax.fori_loop(0, _S // _LN, zbody, 0)

    nach = _AROWS // _ACH

    def _astart(ci, b):
        pltpu.make_async_copy(
            at2.at[pl.ds(arow0 + ci * _ACH, _ACH)],
            abuf.at[b], sems[b]).start()

    _astart(0, 0)
    _astart(1, 1)

    def achunk(c2, _):
        for b in range(2):
            ci = 2 * c2 + b
            pltpu.make_async_copy(
                at2.at[pl.ds(arow0 + ci * _ACH, _ACH)],
                abuf.at[b], sems[b]).wait()

            @plsc.parallel_loop(0, _S // _LN, unroll=4)
            def _(j, b=b):
                vs = [abuf[b, r, pl.ds(j * _LN, _LN)] for r in range(_ACH)]
                aacc[pl.ds(j * _LN, _LN)] = (
                    aacc[pl.ds(j * _LN, _LN)] + _tsum(vs))

            @pl.when(ci + 2 < nach)
            def _(ci=ci, b=b):
                _astart(ci + 2, b)
        return 0

    lax.fori_loop(0, nach // 2, achunk, 0)
    pltpu.sync_copy(aacc, ap_o.at[pl.ds(t * _S, _S)])


_sc_call = functools.partial(
    pl.kernel,
    mesh=plsc.VectorSubcoreMesh(core_axis_name="c", subcore_axis_name="s"),
    compiler_params=pltpu.CompilerParams(needs_layout_passes=False),
    out_type=[
        jax.ShapeDtypeStruct((1, _HD), jnp.float32),
        jax.ShapeDtypeStruct((_LN * _S,), jnp.float32),
        jax.ShapeDtypeStruct((_LN * _S,), jnp.float32),
        jax.ShapeDtypeStruct((_NW * _S,), jnp.float32),
    ],
    scratch_types=[
        pltpu.VMEM((2, _DCH, _HD // _NS), jnp.float32),
        pltpu.VMEM((2, _ECH, _HD), jnp.float32),
        pltpu.VMEM((2, _ACH, _S), jnp.float32),
        pltpu.VMEM((_HD // _NS,), jnp.float32),
        pltpu.VMEM((_HD,), jnp.float32),
        pltpu.VMEM((_LN, _EROWS), jnp.float32),
        pltpu.VMEM((_LN, _EROWS), jnp.float32),
        pltpu.VMEM((_S,), jnp.float32),
        pltpu.VMEM_SHARED((_HD,), jnp.float32),
        pltpu.SemaphoreType.DMA,
        pltpu.SemaphoreType.DMA,
    ],
)(_sc_body)


def _key_i32(x):
    b = lax.bitcast_convert_type(x, jnp.int32)
    return jnp.where(b >= 0, b, jnp.int32(_INT_MIN) - b)


def _final_body(cs_ref, ap_ref, cn_ref, sq_ref, hs_ref, lg_ref, nl_ref,
                cons_ref, acc_ref, core_ref, noise_ref, prune_ref, fl_ref):
    n_attn = jnp.float32(_NLAYERS * 8 * _S)
    colsum = cs_ref[...] + jnp.sum(ap_ref[...], axis=0, keepdims=True)
    agg = colsum / n_attn
    a_norm = agg / (jnp.sum(agg) + 1e-8)
    a_scaled = jnp.clip(a_norm * jnp.float32(_S), 0.0, 1.0)

    h_mean = hs_ref[...] / jnp.float32(_S)
    h_norm = jnp.sqrt(jnp.sum(h_mean * h_mean))
    sq = jnp.sum(sq_ref[...], axis=0, keepdims=True)
    vt_norm = jnp.sqrt(sq)
    vdot = jnp.sum(cn_ref[...], axis=0, keepdims=True) / jnp.float32(_S)
    cos = vdot / (vt_norm + 1e-8) / (h_norm + 1e-8)
    sem = 0.5 * (cos + 1.0)
    consistency = 0.5 * sem + 0.5 * a_scaled

    key = _key_i32(consistency)
    k = jnp.int32(_KEEP)

    def count_ge(c):
        return jnp.sum((key >= c).astype(jnp.int32))

    m0 = jnp.where(count_ge(jnp.int32(0)) >= k, jnp.int32(0),
                   jnp.int32(_INT_MIN))

    def body(j, m):
        bit = lax.shift_left(jnp.int32(1), jnp.int32(30) - j)
        cand = m + bit
        return jnp.where(count_ge(cand) >= k, cand, m)

    m = lax.fori_loop(0, 31, body, m0)

    core = jnp.logical_or(key >= m, consistency >= jnp.float32(_THRESH))
    cons_ref[...] = consistency
    acc_ref[...] = consistency
    core_ref[...] = core
    noise_ref[...] = jnp.logical_not(core)
    prune_ref[...] = core
    fl_ref[...] = (1.0 + _LAM) * lg_ref[...] - _LAM * nl_ref[...]


def kernel(input_embeds, decoder_hidden_states, attn_0, attn_1, attn_2,
           logits, noise_logits):
    B, S, D = input_embeds.shape
    H = attn_0.shape[1]
    V = logits.shape[1]
    rows = H * S

    dec = decoder_hidden_states.reshape(S, D)
    emb = input_embeds.reshape(S, D)
    at2 = attn_2.reshape(rows, S)
    hsum, cosnum, sqnorm, apart = _sc_call(dec, emb, at2)

    bq = 512
    nq = rows // bq
    a0 = attn_0.reshape(rows, S)
    a1 = attn_1.reshape(rows, S)
    off2 = _SC_AROWS // bq
    n2 = (rows - _SC_AROWS) // bq

    colsum = pl.pallas_call(
        functools.partial(_attn_body, n2=n2),
        grid=(nq,),
        in_specs=[
            pl.BlockSpec((bq, S), lambda i: (i, 0)),
            pl.BlockSpec((bq, S), lambda i: (i, 0)),
            pl.BlockSpec((bq, S),
                         lambda i: (off2 + jnp.minimum(i, n2 - 1), 0)),
        ],
        out_specs=pl.BlockSpec((1, S), lambda i: (0, 0)),
        out_shape=jax.ShapeDtypeStruct((1, S), jnp.float32),
    )(a0, a1, at2)

    cons, acc, core, noise, prune, final_logits = pl.pallas_call(
        _final_body,
        out_shape=[
            jax.ShapeDtypeStruct((1, S), jnp.float32),
            jax.ShapeDtypeStruct((1, S), jnp.float32),
            jax.ShapeDtypeStruct((1, S), jnp.bool_),
            jax.ShapeDtypeStruct((1, S), jnp.bool_),
            jax.ShapeDtypeStruct((1, S), jnp.bool_),
            jax.ShapeDtypeStruct((B, V), jnp.float32),
        ],
    )(colsum, apart.reshape(_NW, S), cosnum.reshape(_LN, S),
      sqnorm.reshape(_LN, S), hsum, logits, noise_logits)

    return (cons.reshape(B, S), acc.reshape(B, S), core.reshape(B, S),
            noise.reshape(B, S), prune.reshape(B, S), final_logits)


# speedup vs baseline: 1.0075x; 1.0075x over previous
import functools
import jax
import jax.numpy as jnp
from jax import lax
from jax.experimental import pallas as pl
from jax.experimental.pallas import tpu as pltpu
from jax.experimental.pallas import tpu_sc as plsc

_HD = 4096
_S = 2048
_NLAYERS = 3
_THRESH = 0.5
_KEEP = 1843
_LAM = 1.0
_INT_MIN = -2147483648

_NC = 2
_NS = 16
_LN = 16
_NW = _NC * _NS

_DCH = 64
_ECH = 4
_ACH = 8
_EROWS = _S // _NW
_AROWS = 16
_SC_AROWS = _NW * _AROWS


def _attn_body(a0_ref, a1_ref, a2_ref, cs_ref, *, n2):
    i = pl.program_id(0)

    @pl.when(i == 0)
    def _():
        cs_ref[...] = jnp.zeros_like(cs_ref)
    s = (jnp.sum(a0_ref[...], axis=0, keepdims=True)
         + jnp.sum(a1_ref[...], axis=0, keepdims=True))
    w = (i < n2).astype(jnp.float32)
    cs_ref[...] += s + w * jnp.sum(a2_ref[...], axis=0, keepdims=True)


def _tsum(xs):
    while len(xs) > 1:
        nx = [xs[p] + xs[p + 1] for p in range(0, len(xs) - 1, 2)]
        if len(xs) % 2:
            nx.append(xs[-1])
        xs = nx
    return xs[0]


def _sc_body(dec, emb, at2, hsum_o, cn_o, sq_o, ap_o,
             dbuf, ebuf, abuf, accv, hbuf, cnv, sqv, aacc, shsum,
             sem_a, sem_b):
    cid = lax.axis_index("c")
    sid = lax.axis_index("s")
    t = cid * _NS + sid
    sems = [sem_a, sem_b]

    ncols = _HD // _NS
    col0 = sid * ncols
    nch = _S // _DCH

    def _dstart(ci, b):
        pltpu.make_async_copy(
            dec.at[pl.ds(ci * _DCH, _DCH), pl.ds(col0, ncols)],
            dbuf.at[b], sems[b]).start()

    _dstart(0, 0)
    _dstart(1, 1)
    acc0 = tuple(jnp.zeros((_LN,), jnp.float32) for _ in range(ncols // _LN))

    def dchunk(c2, acc):
        for b in range(2):
            ci = 2 * c2 + b
            pltpu.make_async_copy(
                dec.at[pl.ds(ci * _DCH, _DCH), pl.ds(col0, ncols)],
                dbuf.at[b], sems[b]).wait()

            def rbody(r, a, b=b):
                return tuple(a[j] + dbuf[b, r, pl.ds(j * _LN, _LN)]
                             for j in range(ncols // _LN))
            acc = lax.fori_loop(0, _DCH, rbody, acc)

            @pl.when(ci + 2 < nch)
            def _(ci=ci, b=b):
                _dstart(ci + 2, b)
        return acc

    acc = lax.fori_loop(0, nch // 2, dchunk, acc0)

    for j in range(ncols // _LN):
        accv[pl.ds(j * _LN, _LN)] = acc[j]
    pltpu.sync_copy(accv, shsum.at[pl.ds(col0, ncols)])
    plsc.subcore_barrier()
    pltpu.sync_copy(shsum, hbuf)

    @pl.when(jnp.logical_and(cid == 0, sid == 0))
    def _():
        pltpu.sync_copy(hbuf, hsum_o.at[0])

    erow0 = t * _EROWS
    nech = _EROWS // _ECH
    lane_iota = lax.iota(jnp.int32, _LN)

    def _estart(ci, b):
        pltpu.make_async_copy(
            emb.at[pl.ds(erow0 + ci * _ECH, _ECH)],
            ebuf.at[b], sems[b]).start()

    _estart(0, 0)
    _estart(1, 1)

    def echunk(c2, _):
        for b in range(2):
            ci = 2 * c2 + b
            pltpu.make_async_copy(
                emb.at[pl.ds(erow0 + ci * _ECH, _ECH)],
                ebuf.at[b], sems[b]).wait()

            zc = jnp.zeros((_LN,), jnp.float32)

            @plsc.parallel_loop(0, _HD // 128, unroll=2,
                                carry=(zc,) * (2 * _ECH))
            def vs(bb, cs, b=b):
                out = list(cs)
                h8 = [hbuf[pl.ds(bb * 128 + u * _LN, _LN)] for u in range(8)]
                for r in range(_ECH):
                    ev = [ebuf[b, r, pl.ds(bb * 128 + u * _LN, _LN)]
                          for u in range(8)]
                    out[r] = out[r] + _tsum([ev[u] * h8[u] for u in range(8)])
                    out[_ECH + r] = out[_ECH + r] + _tsum(
                        [ev[u] * ev[u] for u in range(8)])
                return tuple(out)

            for r in range(_ECH):
                rl = jnp.full((_LN,), ci * _ECH + r, jnp.int32)
                plsc.store_scatter(cnv, [lane_iota, rl], vs[r])
                plsc.store_scatter(sqv, [lane_iota, rl], vs[_ECH + r])

            @pl.when(ci + 2 < nech)
            def _(ci=ci, b=b):
                _estart(ci + 2, b)
        return 0

    lax.fori_loop(0, nech // 2, echunk, 0)
    outcps = []
    for j in range(_LN):
        outcps.append(pltpu.make_async_copy(
            cnv.at[j], cn_o.at[pl.ds(j * _S + erow0, _EROWS)], sem_a))
        outcps.append(pltpu.make_async_copy(
            sqv.at[j], sq_o.at[pl.ds(j * _S + erow0, _EROWS)], sem_b))
    for cp in outcps:
        cp.start()
    for cp in outcps:
        cp.wait()

    arow0 = t * _AROWS
    zv = jnp.zeros((_LN,), jnp.float32)

    def zbody(j, _):
        aacc[pl.ds(j * _LN, _LN)] = zv
        return 0
    lax.fori_loop(0, _S // _LN, zbody, 0)

    nach = _AROWS // _ACH

    def _astart(ci, b):
        pltpu.make_async_copy(
            at2.at[pl.ds(arow0 + ci * _ACH, _ACH)],
            abuf.at[b], sems[b]).start()

    _astart(0, 0)
    _astart(1, 1)

    def achunk(c2, _):
        for b in range(2):
            ci = 2 * c2 + b
            pltpu.make_async_copy(
                at2.at[pl.ds(arow0 + ci * _ACH, _ACH)],
                abuf.at[b], sems[b]).wait()

            @plsc.parallel_loop(0, _S // _LN, unroll=4)
            def _(j, b=b):
                vs = [abuf[b, r, pl.ds(j * _LN, _LN)] for r in range(_ACH)]
                aacc[pl.ds(j * _LN, _LN)] = (
                    aacc[pl.ds(j * _LN, _LN)] + _tsum(vs))

            @pl.when(ci + 2 < nach)
            def _(ci=ci, b=b):
                _astart(ci + 2, b)
        return 0

    lax.fori_loop(0, nach // 2, achunk, 0)
    pltpu.sync_copy(aacc, ap_o.at[pl.ds(t * _S, _S)])


_sc_call = functools.partial(
    pl.kernel,
    mesh=plsc.VectorSubcoreMesh(core_axis_name="c", subcore_axis_name="s"),
    compiler_params=pltpu.CompilerParams(needs_layout_passes=False),
    out_type=[
        jax.ShapeDtypeStruct((1, _HD), jnp.float32),
        jax.ShapeDtypeStruct((_LN * _S,), jnp.float32),
        jax.ShapeDtypeStruct((_LN * _S,), jnp.float32),
        jax.ShapeDtypeStruct((_NW * _S,), jnp.float32),
    ],
    scratch_types=[
        pltpu.VMEM((2, _DCH, _HD // _NS), jnp.float32),
        pltpu.VMEM((2, _ECH, _HD), jnp.float32),
        pltpu.VMEM((2, _ACH, _S), jnp.float32),
        pltpu.VMEM((_HD // _NS,), jnp.float32),
        pltpu.VMEM((_HD,), jnp.float32),
        pltpu.VMEM((_LN, _EROWS), jnp.float32),
        pltpu.VMEM((_LN, _EROWS), jnp.float32),
        pltpu.VMEM((_S,), jnp.float32),
        pltpu.VMEM_SHARED((_HD,), jnp.float32),
        pltpu.SemaphoreType.DMA,
        pltpu.SemaphoreType.DMA,
    ],
)(_sc_body)


def _key_i32(x):
    b = lax.bitcast_convert_type(x, jnp.int32)
    return jnp.where(b >= 0, b, jnp.int32(_INT_MIN) - b)


def _final_body(cs_ref, ap_ref, cn_ref, sq_ref, hs_ref, lg_ref, nl_ref,
                cons_ref, acc_ref, core_ref, noise_ref, prune_ref, fl_ref):
    n_attn = jnp.float32(_NLAYERS * 8 * _S)
    colsum = cs_ref[...] + jnp.sum(ap_ref[...], axis=0, keepdims=True)
    agg = colsum / n_attn
    a_norm = agg / (jnp.sum(agg) + 1e-8)
    a_scaled = jnp.clip(a_norm * jnp.float32(_S), 0.0, 1.0)

    h_mean = hs_ref[...] / jnp.float32(_S)
    h_norm = jnp.sqrt(jnp.sum(h_mean * h_mean))
    sq = jnp.sum(sq_ref[...], axis=0, keepdims=True)
    vt_norm = jnp.sqrt(sq)
    vdot = jnp.sum(cn_ref[...], axis=0, keepdims=True) / jnp.float32(_S)
    cos = vdot / (vt_norm + 1e-8) / (h_norm + 1e-8)
    sem = 0.5 * (cos + 1.0)
    consistency = 0.5 * sem + 0.5 * a_scaled

    key = _key_i32(consistency)
    k = jnp.int32(_KEEP)

    def count_ge(c):
        return jnp.sum((key >= c).astype(jnp.int32))

    m0 = jnp.where(count_ge(jnp.int32(0)) >= k, jnp.int32(0),
                   jnp.int32(_INT_MIN))

    def body(j, m):
        bit = lax.shift_left(jnp.int32(1), jnp.int32(30) - j)
        cand = m + bit
        return jnp.where(count_ge(cand) >= k, cand, m)

    m = lax.fori_loop(0, 31, body, m0)

    core = jnp.logical_or(key >= m, consistency >= jnp.float32(_THRESH))
    cons_ref[...] = consistency
    acc_ref[...] = consistency
    core_ref[...] = core
    noise_ref[...] = jnp.logical_not(core)
    prune_ref[...] = core
    fl_ref[...] = (1.0 + _LAM) * lg_ref[...] - _LAM * nl_ref[...]


def kernel(input_embeds, decoder_hidden_states, attn_0, attn_1, attn_2,
           logits, noise_logits):
    B, S, D = input_embeds.shape
    H = attn_0.shape[1]
    V = logits.shape[1]
    rows = H * S

    dec = decoder_hidden_states.reshape(S, D)
    emb = input_embeds.reshape(S, D)
    at2 = attn_2.reshape(rows, S)
    hsum, cosnum, sqnorm, apart = _sc_call(dec, emb, at2)

    bq = 512
    nq = rows // bq
    a0 = attn_0.reshape(rows, S)
    a1 = attn_1.reshape(rows, S)
    off2 = _SC_AROWS // bq
    n2 = (rows - _SC_AROWS) // bq

    colsum = pl.pallas_call(
        functools.partial(_attn_body, n2=n2),
        grid=(nq,),
        in_specs=[
            pl.BlockSpec((bq, S), lambda i: (i, 0)),
            pl.BlockSpec((bq, S), lambda i: (i, 0)),
            pl.BlockSpec((bq, S),
                         lambda i: (off2 + jnp.minimum(i, n2 - 1), 0)),
        ],
        out_specs=pl.BlockSpec((1, S), lambda i: (0, 0)),
        out_shape=jax.ShapeDtypeStruct((1, S), jnp.float32),
    )(a0, a1, at2)

    cons, acc, core, noise, prune, final_logits = pl.pallas_call(
        _final_body,
        out_shape=[
            jax.ShapeDtypeStruct((1, S), jnp.float32),
            jax.ShapeDtypeStruct((1, S), jnp.float32),
            jax.ShapeDtypeStruct((1, S), jnp.bool_),
            jax.ShapeDtypeStruct((1, S), jnp.bool_),
            jax.ShapeDtypeStruct((1, S), jnp.bool_),
            jax.ShapeDtypeStruct((B, V), jnp.float32),
        ],
    )(colsum, apart.reshape(_NW, S), cosnum.reshape(_LN, S),
      sqnorm.reshape(_LN, S), hsum, logits, noise_logits)

    return (cons.reshape(B, S), acc.reshape(B, S), core.reshape(B, S),
            noise.reshape(B, S), prune.reshape(B, S), final_logits)


# speedup vs baseline: 1.0215x; 1.0139x over previous
import functools
import jax
import jax.numpy as jnp
from jax import lax
from jax.experimental import pallas as pl
from jax.experimental.pallas import tpu as pltpu
from jax.experimental.pallas import tpu_sc as plsc

_HD = 4096
_S = 2048
_NLAYERS = 3
_THRESH = 0.5
_KEEP = 1843
_LAM = 1.0
_INT_MIN = -2147483648

_NC = 2
_NS = 16
_LN = 16
_NW = _NC * _NS

_DCH = 64
_ECH = 4
_ACH = 8
_EROWS = _S // _NW
_AROWS = 0
_SC_AROWS = _NW * _AROWS


def _attn_body(a0_ref, a1_ref, a2_ref, cs_ref, *, n2):
    i = pl.program_id(0)

    @pl.when(i == 0)
    def _():
        cs_ref[...] = jnp.zeros_like(cs_ref)
    s = (jnp.sum(a0_ref[...], axis=0, keepdims=True)
         + jnp.sum(a1_ref[...], axis=0, keepdims=True))
    w = (i < n2).astype(jnp.float32)
    cs_ref[...] += s + w * jnp.sum(a2_ref[...], axis=0, keepdims=True)


def _tsum(xs):
    while len(xs) > 1:
        nx = [xs[p] + xs[p + 1] for p in range(0, len(xs) - 1, 2)]
        if len(xs) % 2:
            nx.append(xs[-1])
        xs = nx
    return xs[0]


def _sc_body(dec, emb, at2, hsum_o, cn_o, sq_o, ap_o,
             dbuf, ebuf, abuf, accv, hbuf, cnv, sqv, aacc, shsum,
             sem_a, sem_b):
    cid = lax.axis_index("c")
    sid = lax.axis_index("s")
    t = cid * _NS + sid
    sems = [sem_a, sem_b]

    ncols = _HD // _NS
    col0 = sid * ncols
    nch = _S // _DCH

    def _dstart(ci, b):
        pltpu.make_async_copy(
            dec.at[pl.ds(ci * _DCH, _DCH), pl.ds(col0, ncols)],
            dbuf.at[b], sems[b]).start()

    _dstart(0, 0)
    _dstart(1, 1)
    acc0 = tuple(jnp.zeros((_LN,), jnp.float32) for _ in range(ncols // _LN))

    def dchunk(c2, acc):
        for b in range(2):
            ci = 2 * c2 + b
            pltpu.make_async_copy(
                dec.at[pl.ds(ci * _DCH, _DCH), pl.ds(col0, ncols)],
                dbuf.at[b], sems[b]).wait()

            def rbody(r, a, b=b):
                return tuple(a[j] + dbuf[b, r, pl.ds(j * _LN, _LN)]
                             for j in range(ncols // _LN))
            acc = lax.fori_loop(0, _DCH, rbody, acc)

            @pl.when(ci + 2 < nch)
            def _(ci=ci, b=b):
                _dstart(ci + 2, b)
        return acc

    acc = lax.fori_loop(0, nch // 2, dchunk, acc0)

    for j in range(ncols // _LN):
        accv[pl.ds(j * _LN, _LN)] = acc[j]
    pltpu.sync_copy(accv, shsum.at[pl.ds(col0, ncols)])
    plsc.subcore_barrier()
    pltpu.sync_copy(shsum, hbuf)

    @pl.when(jnp.logical_and(cid == 0, sid == 0))
    def _():
        pltpu.sync_copy(hbuf, hsum_o.at[0])

    erow0 = t * _EROWS
    nech = _EROWS // _ECH
    lane_iota = lax.iota(jnp.int32, _LN)

    def _estart(ci, b):
        pltpu.make_async_copy(
            emb.at[pl.ds(erow0 + ci * _ECH, _ECH)],
            ebuf.at[b], sems[b]).start()

    _estart(0, 0)
    _estart(1, 1)

    def echunk(c2, _):
        for b in range(2):
            ci = 2 * c2 + b
            pltpu.make_async_copy(
                emb.at[pl.ds(erow0 + ci * _ECH, _ECH)],
                ebuf.at[b], sems[b]).wait()

            zc = jnp.zeros((_LN,), jnp.float32)

            @plsc.parallel_loop(0, _HD // 128, unroll=2,
                                carry=(zc,) * (2 * _ECH))
            def vs(bb, cs, b=b):
                out = list(cs)
                h8 = [hbuf[pl.ds(bb * 128 + u * _LN, _LN)] for u in range(8)]
                for r in range(_ECH):
                    ev = [ebuf[b, r, pl.ds(bb * 128 + u * _LN, _LN)]
                          for u in range(8)]
                    out[r] = out[r] + _tsum([ev[u] * h8[u] for u in range(8)])
                    out[_ECH + r] = out[_ECH + r] + _tsum(
                        [ev[u] * ev[u] for u in range(8)])
                return tuple(out)

            for r in range(_ECH):
                rl = jnp.full((_LN,), ci * _ECH + r, jnp.int32)
                plsc.store_scatter(cnv, [lane_iota, rl], vs[r])
                plsc.store_scatter(sqv, [lane_iota, rl], vs[_ECH + r])

            @pl.when(ci + 2 < nech)
            def _(ci=ci, b=b):
                _estart(ci + 2, b)
        return 0

    lax.fori_loop(0, nech // 2, echunk, 0)
    outcps = []
    for j in range(_LN):
        outcps.append(pltpu.make_async_copy(
            cnv.at[j], cn_o.at[pl.ds(j * _S + erow0, _EROWS)], sem_a))
        outcps.append(pltpu.make_async_copy(
            sqv.at[j], sq_o.at[pl.ds(j * _S + erow0, _EROWS)], sem_b))
    for cp in outcps:
        cp.start()
    for cp in outcps:
        cp.wait()

    if not _AROWS:
        return
    arow0 = t * _AROWS
    zv = jnp.zeros((_LN,), jnp.float32)

    def zbody(j, _):
        aacc[pl.ds(j * _LN, _LN)] = zv
        return 0
    lax.fori_loop(0, _S // _LN, zbody, 0)

    nach = _AROWS // _ACH

    def _astart(ci, b):
        pltpu.make_async_copy(
            at2.at[pl.ds(arow0 + ci * _ACH, _ACH)],
            abuf.at[b], sems[b]).start()

    _astart(0, 0)
    _astart(1, 1)

    def achunk(c2, _):
        for b in range(2):
            ci = 2 * c2 + b
            pltpu.make_async_copy(
                at2.at[pl.ds(arow0 + ci * _ACH, _ACH)],
                abuf.at[b], sems[b]).wait()

            @plsc.parallel_loop(0, _S // _LN, unroll=4)
            def _(j, b=b):
                vs = [abuf[b, r, pl.ds(j * _LN, _LN)] for r in range(_ACH)]
                aacc[pl.ds(j * _LN, _LN)] = (
                    aacc[pl.ds(j * _LN, _LN)] + _tsum(vs))

            @pl.when(ci + 2 < nach)
            def _(ci=ci, b=b):
                _astart(ci + 2, b)
        return 0

    lax.fori_loop(0, nach // 2, achunk, 0)
    pltpu.sync_copy(aacc, ap_o.at[pl.ds(t * _S, _S)])


_sc_call = functools.partial(
    pl.kernel,
    mesh=plsc.VectorSubcoreMesh(core_axis_name="c", subcore_axis_name="s"),
    compiler_params=pltpu.CompilerParams(needs_layout_passes=False),
    out_type=[
        jax.ShapeDtypeStruct((1, _HD), jnp.float32),
        jax.ShapeDtypeStruct((_LN * _S,), jnp.float32),
        jax.ShapeDtypeStruct((_LN * _S,), jnp.float32),
        jax.ShapeDtypeStruct((_NW * _S,), jnp.float32),
    ],
    scratch_types=[
        pltpu.VMEM((2, _DCH, _HD // _NS), jnp.float32),
        pltpu.VMEM((2, _ECH, _HD), jnp.float32),
        pltpu.VMEM((2, _ACH, _S), jnp.float32),
        pltpu.VMEM((_HD // _NS,), jnp.float32),
        pltpu.VMEM((_HD,), jnp.float32),
        pltpu.VMEM((_LN, _EROWS), jnp.float32),
        pltpu.VMEM((_LN, _EROWS), jnp.float32),
        pltpu.VMEM((_S,), jnp.float32),
        pltpu.VMEM_SHARED((_HD,), jnp.float32),
        pltpu.SemaphoreType.DMA,
        pltpu.SemaphoreType.DMA,
    ],
)(_sc_body)


def _key_i32(x):
    b = lax.bitcast_convert_type(x, jnp.int32)
    return jnp.where(b >= 0, b, jnp.int32(_INT_MIN) - b)


def _final_body(cs_ref, cn_ref, sq_ref, hs_ref, lg_ref, nl_ref,
                cons_ref, acc_ref, core_ref, noise_ref, prune_ref, fl_ref):
    n_attn = jnp.float32(_NLAYERS * 8 * _S)
    agg = cs_ref[...] / n_attn
    a_norm = agg / (jnp.sum(agg) + 1e-8)
    a_scaled = jnp.clip(a_norm * jnp.float32(_S), 0.0, 1.0)

    h_mean = hs_ref[...] / jnp.float32(_S)
    h_norm = jnp.sqrt(jnp.sum(h_mean * h_mean))
    sq = jnp.sum(sq_ref[...], axis=0, keepdims=True)
    vt_norm = jnp.sqrt(sq)
    vdot = jnp.sum(cn_ref[...], axis=0, keepdims=True) / jnp.float32(_S)
    cos = vdot / (vt_norm + 1e-8) / (h_norm + 1e-8)
    sem = 0.5 * (cos + 1.0)
    consistency = 0.5 * sem + 0.5 * a_scaled

    key = _key_i32(consistency)
    k = jnp.int32(_KEEP)

    def count_ge(c):
        return jnp.sum((key >= c).astype(jnp.int32))

    m0 = jnp.where(count_ge(jnp.int32(0)) >= k, jnp.int32(0),
                   jnp.int32(_INT_MIN))

    def body(j, m):
        bit = lax.shift_left(jnp.int32(1), jnp.int32(30) - j)
        cand = m + bit
        return jnp.where(count_ge(cand) >= k, cand, m)

    m = lax.fori_loop(0, 31, body, m0)

    core = jnp.logical_or(key >= m, consistency >= jnp.float32(_THRESH))
    cons_ref[...] = consistency
    acc_ref[...] = consistency
    core_ref[...] = core
    noise_ref[...] = jnp.logical_not(core)
    prune_ref[...] = core
    fl_ref[...] = (1.0 + _LAM) * lg_ref[...] - _LAM * nl_ref[...]


def kernel(input_embeds, decoder_hidden_states, attn_0, attn_1, attn_2,
           logits, noise_logits):
    B, S, D = input_embeds.shape
    H = attn_0.shape[1]
    V = logits.shape[1]
    rows = H * S

    dec = decoder_hidden_states.reshape(S, D)
    emb = input_embeds.reshape(S, D)
    at2 = attn_2.reshape(rows, S)
    hsum, cosnum, sqnorm, apart = _sc_call(dec, emb, at2)

    bq = 512
    nq = rows // bq
    a0 = attn_0.reshape(rows, S)
    a1 = attn_1.reshape(rows, S)
    off2 = _SC_AROWS // bq
    n2 = (rows - _SC_AROWS) // bq

    colsum = pl.pallas_call(
        functools.partial(_attn_body, n2=n2),
        grid=(nq,),
        in_specs=[
            pl.BlockSpec((bq, S), lambda i: (i, 0)),
            pl.BlockSpec((bq, S), lambda i: (i, 0)),
            pl.BlockSpec((bq, S),
                         lambda i: (off2 + jnp.minimum(i, n2 - 1), 0)),
        ],
        out_specs=pl.BlockSpec((1, S), lambda i: (0, 0)),
        out_shape=jax.ShapeDtypeStruct((1, S), jnp.float32),
    )(a0, a1, at2)

    cons, acc, core, noise, prune, final_logits = pl.pallas_call(
        _final_body,
        out_shape=[
            jax.ShapeDtypeStruct((1, S), jnp.float32),
            jax.ShapeDtypeStruct((1, S), jnp.float32),
            jax.ShapeDtypeStruct((1, S), jnp.bool_),
            jax.ShapeDtypeStruct((1, S), jnp.bool_),
            jax.ShapeDtypeStruct((1, S), jnp.bool_),
            jax.ShapeDtypeStruct((B, V), jnp.float32),
        ],
    )(colsum, cosnum.reshape(_LN, S), sqnorm.reshape(_LN, S),
      hsum, logits, noise_logits)

    return (cons.reshape(B, S), acc.reshape(B, S), core.reshape(B, S),
            noise.reshape(B, S), prune.reshape(B, S), final_logits)
